# baseline (speedup 1.0000x reference)
import functools

import jax
import jax.numpy as jnp
from jax import lax
from jax.experimental import pallas as pl
from jax.experimental.pallas import tpu as pltpu
from jax.experimental.pallas import tpu_sc as plsc

ROW = 64
NC, NS, L = 2, 16, 16
NW = NC * NS

N = 4096 * 50
PW = N // NW
SLOT = 128
NSLOT = PW // SLOT


def _sc_gather(dt_flat, W_day, W_time):
    mesh = plsc.VectorSubcoreMesh(core_axis_name="c", subcore_axis_name="s")

    @functools.partial(
        pl.kernel,
        out_type=jax.ShapeDtypeStruct((N, 2 * ROW), jnp.float32),
        mesh=mesh,
        scratch_types=[
            pltpu.VMEM((2 * PW,), jnp.int32),
            pltpu.VMEM((NSLOT, SLOT), jnp.int32),
            pltpu.VMEM((NSLOT, SLOT), jnp.int32),
            pltpu.VMEM((SLOT, ROW), jnp.float32),
            pltpu.VMEM((SLOT, ROW), jnp.float32),
            pltpu.VMEM((SLOT, ROW), jnp.float32),
            pltpu.VMEM((SLOT, ROW), jnp.float32),
            pltpu.SemaphoreType.DMA,
            pltpu.SemaphoreType.DMA,
        ],
        compiler_params=pltpu.CompilerParams(use_tc_tiling_on_sc=False),
    )
    def k(dt_hbm, wd_hbm, wt_hbm, out_hbm,
          rawbuf, tidx, didx, ta, da, tb, db, gsem, wsem):
        wid = lax.axis_index("s") * NC + lax.axis_index("c")
        base = wid * PW
        lane = lax.iota(jnp.int32, L)
        half = lane < (L // 2)
        todd = (2 * lane + 1) & (L - 1)
        deven = (2 * lane) & (L - 1)

        pltpu.sync_copy(dt_hbm.at[pl.ds(2 * base, 2 * PW)], rawbuf)

        def deint_body(s, _):
            for u in range(SLOT // L):
                off = s * 2 * SLOT + u * 2 * L
                v0 = rawbuf[pl.ds(off, L)]
                v1 = rawbuf[pl.ds(off + L, L)]
                t = jnp.where(half,
                              v0.at[todd].get(mode="promise_in_bounds"),
                              v1.at[todd].get(mode="promise_in_bounds"))
                d = jnp.where(half,
                              v0.at[deven].get(mode="promise_in_bounds"),
                              v1.at[deven].get(mode="promise_in_bounds"))
                tidx[s, pl.ds(u * L, L)] = t
                didx[s, pl.ds(u * L, L)] = d
            return _

        lax.fori_loop(0, NSLOT, deint_body, None)

        def fire_gathers(s, tbuf, dbuf):
            pltpu.async_copy(wt_hbm.at[tidx.at[s]], tbuf, gsem)
            pltpu.async_copy(wd_hbm.at[didx.at[s]], dbuf, gsem)

        def drain_gathers(tbuf, dbuf):
            pltpu.make_async_copy(wt_hbm.at[tidx.at[0]], tbuf, gsem).wait()
            pltpu.make_async_copy(wd_hbm.at[didx.at[0]], dbuf, gsem).wait()

        def out_t(s):
            return out_hbm.at[pl.ds(base + s * SLOT, SLOT), pl.ds(0, ROW)]

        def out_d(s):
            return out_hbm.at[pl.ds(base + s * SLOT, SLOT), pl.ds(ROW, ROW)]

        fire_gathers(0, ta, da)

        def outer(s2, _):
            for b in range(2):
                tbuf, dbuf = (ta, da) if b == 0 else (tb, db)
                obuf_t, obuf_d = (tb, db) if b == 0 else (ta, da)
                s = s2 * 2 + b
                drain_gathers(tbuf, dbuf)

                @pl.when(s > 0)
                def _():
                    pltpu.make_async_copy(obuf_t, out_t(s - 1), wsem).wait()
                    pltpu.make_async_copy(obuf_d, out_d(s - 1), wsem).wait()

                pltpu.async_copy(tbuf, out_t(s), wsem)
                pltpu.async_copy(dbuf, out_d(s), wsem)

                @pl.when(s < NSLOT - 1)
                def _():
                    fire_gathers(s + 1, obuf_t, obuf_d)

            return _

        lax.fori_loop(0, NSLOT // 2, outer, None)
        pltpu.make_async_copy(tb, out_t(NSLOT - 1), wsem).wait()
        pltpu.make_async_copy(db, out_d(NSLOT - 1), wsem).wait()

    return k(dt_flat, W_day, W_time)


def kernel(daytime, W_day, W_time):
    b, s, _ = daytime.shape
    dt_flat = daytime.astype(jnp.int32).reshape(2 * b * s)
    out2 = _sc_gather(dt_flat, W_day, W_time)
    return out2.reshape(b, s, 2 * ROW)


# speedup vs baseline: 1.5020x; 1.5020x over previous
import functools

import jax
import jax.numpy as jnp
from jax import lax
from jax.experimental import pallas as pl
from jax.experimental.pallas import tpu as pltpu
from jax.experimental.pallas import tpu_sc as plsc

ROW = 64
NC, NS, L = 2, 16, 16
NW = NC * NS

B = 4096
S = 50
PB = B // NW
PW = PB * 2 * S
SLOT = PB


def _sc_gather(dt_flat, W_day, W_time):
    mesh = plsc.VectorSubcoreMesh(core_axis_name="c", subcore_axis_name="s")

    @functools.partial(
        pl.kernel,
        out_type=jax.ShapeDtypeStruct((S, B, 2 * ROW), jnp.float32),
        mesh=mesh,
        scratch_types=[
            pltpu.VMEM((PW,), jnp.int32),
            pltpu.VMEM((S, SLOT), jnp.int32),
            pltpu.VMEM((S, SLOT), jnp.int32),
            pltpu.VMEM((SLOT, ROW), jnp.float32),
            pltpu.VMEM((SLOT, ROW), jnp.float32),
            pltpu.VMEM((SLOT, ROW), jnp.float32),
            pltpu.VMEM((SLOT, ROW), jnp.float32),
            pltpu.SemaphoreType.DMA,
            pltpu.SemaphoreType.DMA,
        ],
        compiler_params=pltpu.CompilerParams(
            use_tc_tiling_on_sc=False,
            needs_layout_passes=False,
        ),
    )
    def k(dt_hbm, wd_hbm, wt_hbm, out_hbm,
          rawbuf, tidx, didx, ta, da, tb, db, gsem, wsem):
        wid = lax.axis_index("s") * NC + lax.axis_index("c")
        b0 = wid * PB
        lane = lax.iota(jnp.int32, L)

        pltpu.sync_copy(dt_hbm.at[pl.ds(wid * PW, PW)], rawbuf)

        def deint_body(s, _):
            for u in range(SLOT // L):
                pos = (u * L + lane) * (2 * S) + 2 * s
                d = plsc.load_gather(rawbuf, [pos])
                t = plsc.load_gather(rawbuf, [pos + 1])
                tidx[s, pl.ds(u * L, L)] = t
                didx[s, pl.ds(u * L, L)] = d
            return _

        lax.fori_loop(0, S, deint_body, None)

        def fire_gathers(s, tbuf, dbuf):
            pltpu.async_copy(wt_hbm.at[tidx.at[s]], tbuf, gsem)
            pltpu.async_copy(wd_hbm.at[didx.at[s]], dbuf, gsem)

        def drain_gathers(tbuf, dbuf):
            pltpu.make_async_copy(wt_hbm.at[tidx.at[0]], tbuf, gsem).wait()
            pltpu.make_async_copy(wd_hbm.at[didx.at[0]], dbuf, gsem).wait()

        def out_t(s):
            return out_hbm.at[s, pl.ds(b0, SLOT), pl.ds(0, ROW)]

        def out_d(s):
            return out_hbm.at[s, pl.ds(b0, SLOT), pl.ds(ROW, ROW)]

        fire_gathers(0, ta, da)

        def outer(s2, _):
            for i in range(2):
                tbuf, dbuf = (ta, da) if i == 0 else (tb, db)
                obuf_t, obuf_d = (tb, db) if i == 0 else (ta, da)
                s = s2 * 2 + i
                drain_gathers(tbuf, dbuf)

                @pl.when(s > 0)
                def _():
                    pltpu.make_async_copy(obuf_t, out_t(s - 1), wsem).wait()
                    pltpu.make_async_copy(obuf_d, out_d(s - 1), wsem).wait()

                pltpu.async_copy(tbuf, out_t(s), wsem)
                pltpu.async_copy(dbuf, out_d(s), wsem)

                @pl.when(s < S - 1)
                def _():
                    fire_gathers(s + 1, obuf_t, obuf_d)

            return _

        lax.fori_loop(0, S // 2, outer, None)
        pltpu.make_async_copy(tb, out_t(S - 1), wsem).wait()
        pltpu.make_async_copy(db, out_d(S - 1), wsem).wait()

    return k(dt_flat, W_day, W_time)


def kernel(daytime, W_day, W_time):
    if daytime.dtype != jnp.int32:
        daytime = daytime.astype(jnp.int32)
    dt_flat = daytime.reshape(2 * B * S)
    out3 = _sc_gather(dt_flat, W_day, W_time)
    return out3.transpose(1, 0, 2)
